# baseline (speedup 1.0000x reference)
import functools

import jax
import jax.numpy as jnp
from jax import lax
from jax.experimental import pallas as pl
from jax.experimental.pallas import tpu as pltpu
from jax.experimental.pallas import tpu_sc as plsc

N = 10000
E = 320000
D = 128
H = 128
EPS = 1e-5

NC = 2
NS = 16
NW = NC * NS
L = 16

N_PAD = 10112
RPT = N_PAD // NS
EPW = E // NW
K = 128
NBUF = 2
IB = 16
ITERS = IB * (-(-E // (NW * K * IB)))
E_PAD = NW * K * ITERS

_mesh = plsc.VectorSubcoreMesh(core_axis_name="c", subcore_axis_name="s")


@functools.partial(
    pl.kernel,
    out_type=jax.ShapeDtypeStruct((NW, N_PAD), jnp.float32),
    mesh=_mesh,
    scratch_types=[
        pltpu.VMEM((EPW,), jnp.int32),
        pltpu.VMEM((N_PAD,), jnp.float32),
    ],
    compiler_params=pltpu.CompilerParams(needs_layout_passes=False),
)
def _deg_kernel(dst_hbm, out_hbm, dst_v, hist_v):
    c = lax.axis_index("c")
    s = lax.axis_index("s")
    wid = s * NC + c
    pltpu.sync_copy(dst_hbm.at[pl.ds(wid * EPW, EPW)], dst_v)

    zeros = jnp.zeros((L,), jnp.float32)

    def zbody(i, carry):
        hist_v[pl.ds(i * L, L)] = zeros
        return carry

    lax.fori_loop(0, N_PAD // L, zbody, 0)

    ones = jnp.ones((L,), jnp.float32)

    def body(i, carry):
        idx = dst_v[pl.ds(i * L, L)]
        plsc.addupdate_scatter(hist_v, [idx], ones)
        return carry

    lax.fori_loop(0, EPW // L, body, 0)
    pltpu.sync_copy(hist_v, out_hbm.at[wid])


@functools.partial(
    pl.kernel,
    out_type=jax.ShapeDtypeStruct((NC, N_PAD, H), jnp.float32),
    mesh=_mesh,
    scratch_types=[
        pltpu.VMEM_SHARED((N_PAD, H), jnp.float32),
        pltpu.VMEM((IB, K), jnp.int32),
        pltpu.VMEM((IB, K), jnp.int32),
        pltpu.VMEM((NBUF, K, H), jnp.float32),
        pltpu.SemaphoreType.DMA,
        pltpu.SemaphoreType.DMA,
        pltpu.SemaphoreType.DMA,
        pltpu.SemaphoreType.DMA,
    ],
    compiler_params=pltpu.CompilerParams(needs_layout_passes=False,
                                         use_tc_tiling_on_sc=False),
)
def _agg_kernel(src_hbm, dst_hbm, y_hbm, zrows_hbm, out_hbm,
                acc_sh, src_v, dst_v, rows_v, sem0, sem1, sem2, sem3):
    c = lax.axis_index("c")
    s = lax.axis_index("s")
    wid = s * NC + c
    sems = [sem0, sem1, sem2, sem3]
    GI = IB // NBUF

    pltpu.sync_copy(zrows_hbm.at[pl.ds(s * RPT, RPT)],
                    acc_sh.at[pl.ds(s * RPT, RPT)])
    plsc.subcore_barrier()

    def block(blk, carry):
        pltpu.sync_copy(src_hbm.at[wid, pl.ds(blk * IB, IB)], src_v)
        pltpu.sync_copy(dst_hbm.at[wid, pl.ds(blk * IB, IB)], dst_v)

        for b in range(NBUF):
            pltpu.async_copy(y_hbm.at[src_v.at[b]], rows_v.at[b], sems[b])

        def inner(g, carry2):
            for b in range(NBUF):
                j = g * NBUF + b
                pltpu.make_async_copy(y_hbm.at[src_v.at[j]],
                                      rows_v.at[b], sems[b]).wait()
                pltpu.sync_copy(rows_v.at[b], acc_sh.at[dst_v.at[j]],
                                add=True)

                @pl.when(g < GI - 1)
                def _():
                    pltpu.async_copy(y_hbm.at[src_v.at[j + NBUF]],
                                     rows_v.at[b], sems[b])
            return carry2

        lax.fori_loop(0, GI, inner, 0)
        return carry

    lax.fori_loop(0, ITERS // IB, block, 0)
    plsc.subcore_barrier()

    pltpu.sync_copy(acc_sh.at[pl.ds(s * RPT, RPT)],
                    out_hbm.at[c, pl.ds(s * RPT, RPT)])


def _xw_body(x_ref, w_ref, xw_ref):
    xw_ref[...] = jnp.dot(x_ref[...], w_ref[...],
                          preferred_element_type=jnp.float32)


def _prescale_body(xw_ref, parts_ref, y_ref, dis_ref):
    deg = jnp.sum(parts_ref[...], axis=0)
    dis = jnp.where(deg > 0, lax.rsqrt(deg), 0.0)
    dis_col = dis.reshape(N_PAD, 1)
    dis_ref[...] = dis_col
    y_ref[...] = dis_col * xw_ref[...]


def _nan_to_num(h):
    h = jnp.where(jnp.isnan(h), 0.0, h)
    h = jnp.where(h == jnp.inf, 100.0, h)
    h = jnp.where(h == -jnp.inf, -100.0, h)
    return h


def _mid_body(p_ref, dis_ref, x_ref, b1_ref, g1_ref, be1_ref, w2_ref,
              h1_ref, y2_ref):
    conv = dis_ref[...] * (p_ref[0] + p_ref[1]) + b1_ref[...]
    bnscale = 1.0 / jnp.sqrt(1.0 + EPS)
    h = conv * bnscale * g1_ref[...] + be1_ref[...]
    h = jax.nn.relu(h) + x_ref[...]
    h = _nan_to_num(h)
    h1_ref[...] = h
    y2_ref[...] = dis_ref[...] * jnp.dot(h, w2_ref[...],
                                         preferred_element_type=jnp.float32)


def _head_body(q_ref, dis_ref, h1_ref, b2_ref, g2_ref, be2_ref,
               wc1_ref, bc1_ref, wc2_ref, bc2_ref, logits_ref, probs_ref):
    conv = dis_ref[...] * (q_ref[0] + q_ref[1]) + b2_ref[...]
    bnscale = 1.0 / jnp.sqrt(1.0 + EPS)
    h = conv * bnscale * g2_ref[...] + be2_ref[...]
    h = h + h1_ref[...]
    h = _nan_to_num(h)
    hid = jax.nn.relu(jnp.dot(h, wc1_ref[...],
                              preferred_element_type=jnp.float32) + bc1_ref[...])
    logits = jnp.dot(hid, wc2_ref[...],
                     preferred_element_type=jnp.float32) + bc2_ref[...]
    logits_ref[...] = logits
    m = jnp.max(logits, axis=1, keepdims=True)
    e = jnp.exp(logits - m)
    probs_ref[...] = e / jnp.sum(e, axis=1, keepdims=True)


def kernel(x, edge_indices, W1, b1, g1, be1, W2, b2, g2, be2, Wc1, bc1, Wc2, bc2):
    edge_index = edge_indices[0]
    src = edge_index[0]
    dst = edge_index[1]

    x_pad = jnp.pad(x, ((0, N_PAD - N), (0, 0)))
    pad_e = E_PAD - E
    pad_idx = (N + jnp.arange(pad_e, dtype=jnp.int32) % (N_PAD - N)).astype(jnp.int32)
    srcp = jnp.concatenate([src, pad_idx])
    dstp = jnp.concatenate([dst, pad_idx])
    srcp = srcp.reshape(NW, ITERS, K)
    dstp = dstp.reshape(NW, ITERS, K)
    zrows = jnp.zeros((N_PAD, H), jnp.float32)

    parts = _deg_kernel(dst)

    xw1 = pl.pallas_call(
        _xw_body,
        out_shape=jax.ShapeDtypeStruct((N_PAD, H), jnp.float32),
    )(x_pad, W1)

    y1, dis_col = pl.pallas_call(
        _prescale_body,
        out_shape=(jax.ShapeDtypeStruct((N_PAD, H), jnp.float32),
                   jax.ShapeDtypeStruct((N_PAD, 1), jnp.float32)),
    )(xw1, parts)

    p = _agg_kernel(srcp, dstp, y1, zrows)

    h1, y2 = pl.pallas_call(
        _mid_body,
        out_shape=(jax.ShapeDtypeStruct((N_PAD, H), jnp.float32),
                   jax.ShapeDtypeStruct((N_PAD, H), jnp.float32)),
    )(p, dis_col, x_pad, b1, g1, be1, W2)

    q = _agg_kernel(srcp, dstp, y2, zrows)

    logits, probs = pl.pallas_call(
        _head_body,
        out_shape=(jax.ShapeDtypeStruct((N_PAD, 2), jnp.float32),
                   jax.ShapeDtypeStruct((N_PAD, 2), jnp.float32)),
    )(q, dis_col, h1, b2, g2, be2, Wc1, bc1, Wc2, bc2)

    return logits[:N], probs[:N]


# speedup vs baseline: 1.0723x; 1.0723x over previous
import functools

import jax
import jax.numpy as jnp
from jax import lax
from jax.experimental import pallas as pl
from jax.experimental.pallas import tpu as pltpu
from jax.experimental.pallas import tpu_sc as plsc

N = 10000
E = 320000
D = 128
H = 128
EPS = 1e-5

NC = 2
NS = 16
NW = NC * NS
L = 16

N_PAD = 10112
RPT = N_PAD // NS
EPW = E // NW
K = 128
NBUF = 2
IB = 16
ITERS = IB * (-(-E // (NW * K * IB)))
E_PAD = NW * K * ITERS

_mesh = plsc.VectorSubcoreMesh(core_axis_name="c", subcore_axis_name="s")


@functools.partial(
    pl.kernel,
    out_type=jax.ShapeDtypeStruct((NW, N_PAD), jnp.float32),
    mesh=_mesh,
    scratch_types=[
        pltpu.VMEM((EPW,), jnp.int32),
        pltpu.VMEM((N_PAD,), jnp.float32),
    ],
    compiler_params=pltpu.CompilerParams(needs_layout_passes=False),
)
def _deg_kernel(dst_hbm, out_hbm, dst_v, hist_v):
    c = lax.axis_index("c")
    s = lax.axis_index("s")
    wid = s * NC + c
    pltpu.sync_copy(dst_hbm.at[pl.ds(wid * EPW, EPW)], dst_v)

    zeros = jnp.zeros((L,), jnp.float32)

    def zbody(i, carry):
        hist_v[pl.ds(i * L, L)] = zeros
        return carry

    lax.fori_loop(0, N_PAD // L, zbody, 0)

    ones = jnp.ones((L,), jnp.float32)

    def body(i, carry):
        idx = dst_v[pl.ds(i * L, L)]
        plsc.addupdate_scatter(hist_v, [idx], ones)
        return carry

    lax.fori_loop(0, EPW // L, body, 0)
    pltpu.sync_copy(hist_v, out_hbm.at[wid])


@functools.partial(
    pl.kernel,
    out_type=jax.ShapeDtypeStruct((NC, N_PAD, H), jnp.float32),
    mesh=_mesh,
    scratch_types=[
        pltpu.VMEM_SHARED((N_PAD, H), jnp.float32),
        pltpu.VMEM((ITERS, K), jnp.int32),
        pltpu.VMEM((2, IB, K), jnp.int32),
        pltpu.VMEM((NBUF, K, H), jnp.float32),
        pltpu.SemaphoreType.DMA,
        pltpu.SemaphoreType.DMA,
        pltpu.SemaphoreType.DMA,
        pltpu.SemaphoreType.DMA,
    ],
    compiler_params=pltpu.CompilerParams(needs_layout_passes=False,
                                         use_tc_tiling_on_sc=False),
)
def _agg_kernel(src_hbm, dst_hbm, y_hbm, zrows_hbm, out_hbm,
                acc_sh, src_v, dst_v, rows_v, gsem0, gsem1, dsem0, dsem1):
    c = lax.axis_index("c")
    s = lax.axis_index("s")
    wid = s * NC + c
    gsems = [gsem0, gsem1]
    dsems = [dsem0, dsem1]
    GI = IB // NBUF
    NBLK = ITERS // IB

    pltpu.sync_copy(zrows_hbm.at[pl.ds(s * RPT, RPT)],
                    acc_sh.at[pl.ds(s * RPT, RPT)])
    pltpu.sync_copy(src_hbm.at[wid], src_v)
    for blk in range(min(2, NBLK)):
        pltpu.async_copy(dst_hbm.at[wid, pl.ds(blk * IB, IB)],
                         dst_v.at[blk], dsems[blk])
    plsc.subcore_barrier()

    for b in range(NBUF):
        pltpu.async_copy(y_hbm.at[src_v.at[b]], rows_v.at[b], gsems[b])

    for blk in range(NBLK):
        buf = blk % 2
        pltpu.make_async_copy(dst_hbm.at[wid, pl.ds(blk * IB, IB)],
                              dst_v.at[buf], dsems[buf]).wait()

        def inner(g, carry2, blk=blk, buf=buf):
            for b in range(NBUF):
                j = blk * IB + g * NBUF + b
                pltpu.make_async_copy(y_hbm.at[src_v.at[j]],
                                      rows_v.at[b], gsems[b]).wait()
                pltpu.sync_copy(rows_v.at[b],
                                acc_sh.at[dst_v.at[buf].at[g * NBUF + b]],
                                add=True)

                @pl.when(j + NBUF < ITERS)
                def _():
                    pltpu.async_copy(y_hbm.at[src_v.at[j + NBUF]],
                                     rows_v.at[b], gsems[b])
            return carry2

        lax.fori_loop(0, GI, inner, 0)
        if blk + 2 < NBLK:
            pltpu.async_copy(dst_hbm.at[wid, pl.ds((blk + 2) * IB, IB)],
                             dst_v.at[buf], dsems[buf])
    plsc.subcore_barrier()

    pltpu.sync_copy(acc_sh.at[pl.ds(s * RPT, RPT)],
                    out_hbm.at[c, pl.ds(s * RPT, RPT)])


def _prescale_body(x_ref, w_ref, parts_ref, y_ref, dis_ref):
    deg = jnp.sum(parts_ref[...], axis=0)
    dis = jnp.where(deg > 0, lax.rsqrt(deg), 0.0)
    dis_col = dis.reshape(N_PAD, 1)
    dis_ref[...] = dis_col
    xw = jnp.dot(x_ref[...], w_ref[...], preferred_element_type=jnp.float32)
    y_ref[...] = dis_col * xw


def _nan_to_num(h):
    h = jnp.where(jnp.isnan(h), 0.0, h)
    h = jnp.where(h == jnp.inf, 100.0, h)
    h = jnp.where(h == -jnp.inf, -100.0, h)
    return h


def _mid_body(p_ref, dis_ref, x_ref, b1_ref, g1_ref, be1_ref, w2_ref,
              h1_ref, y2_ref):
    conv = dis_ref[...] * (p_ref[0] + p_ref[1]) + b1_ref[...]
    bnscale = 1.0 / jnp.sqrt(1.0 + EPS)
    h = conv * bnscale * g1_ref[...] + be1_ref[...]
    h = jax.nn.relu(h) + x_ref[...]
    h = _nan_to_num(h)
    h1_ref[...] = h
    y2_ref[...] = dis_ref[...] * jnp.dot(h, w2_ref[...],
                                         preferred_element_type=jnp.float32)


def _head_body(q_ref, dis_ref, h1_ref, b2_ref, g2_ref, be2_ref,
               wc1_ref, bc1_ref, wc2_ref, bc2_ref, logits_ref, probs_ref):
    conv = dis_ref[...] * (q_ref[0] + q_ref[1]) + b2_ref[...]
    bnscale = 1.0 / jnp.sqrt(1.0 + EPS)
    h = conv * bnscale * g2_ref[...] + be2_ref[...]
    h = h + h1_ref[...]
    h = _nan_to_num(h)
    hid = jax.nn.relu(jnp.dot(h, wc1_ref[...],
                              preferred_element_type=jnp.float32) + bc1_ref[...])
    logits = jnp.dot(hid, wc2_ref[...],
                     preferred_element_type=jnp.float32) + bc2_ref[...]
    logits_ref[...] = logits
    m = jnp.max(logits, axis=1, keepdims=True)
    e = jnp.exp(logits - m)
    probs_ref[...] = e / jnp.sum(e, axis=1, keepdims=True)


def kernel(x, edge_indices, W1, b1, g1, be1, W2, b2, g2, be2, Wc1, bc1, Wc2, bc2):
    edge_index = edge_indices[0]
    src = edge_index[0]
    dst = edge_index[1]

    x_pad = jnp.pad(x, ((0, N_PAD - N), (0, 0)))
    pad_e = E_PAD - E
    pad_idx = (N + jnp.arange(pad_e, dtype=jnp.int32) % (N_PAD - N)).astype(jnp.int32)
    srcp = jnp.concatenate([src, pad_idx])
    dstp = jnp.concatenate([dst, pad_idx])
    srcp = srcp.reshape(NW, ITERS, K)
    dstp = dstp.reshape(NW, ITERS, K)
    zrows = jnp.zeros((N_PAD, H), jnp.float32)

    parts = _deg_kernel(dst)

    y1, dis_col = pl.pallas_call(
        _prescale_body,
        out_shape=(jax.ShapeDtypeStruct((N_PAD, H), jnp.float32),
                   jax.ShapeDtypeStruct((N_PAD, 1), jnp.float32)),
    )(x_pad, W1, parts)

    p = _agg_kernel(srcp, dstp, y1, zrows)

    h1, y2 = pl.pallas_call(
        _mid_body,
        out_shape=(jax.ShapeDtypeStruct((N_PAD, H), jnp.float32),
                   jax.ShapeDtypeStruct((N_PAD, H), jnp.float32)),
    )(p, dis_col, x_pad, b1, g1, be1, W2)

    q = _agg_kernel(srcp, dstp, y2, zrows)

    logits, probs = pl.pallas_call(
        _head_body,
        out_shape=(jax.ShapeDtypeStruct((N_PAD, 2), jnp.float32),
                   jax.ShapeDtypeStruct((N_PAD, 2), jnp.float32)),
    )(q, dis_col, h1, b2, g2, be2, Wc1, bc1, Wc2, bc2)

    return logits[:N], probs[:N]


# speedup vs baseline: 1.0785x; 1.0057x over previous
import functools

import jax
import jax.numpy as jnp
from jax import lax
from jax.experimental import pallas as pl
from jax.experimental.pallas import tpu as pltpu
from jax.experimental.pallas import tpu_sc as plsc

N = 10000
E = 320000
D = 128
H = 128
EPS = 1e-5

NC = 2
NS = 16
NW = NC * NS
L = 16

N_PAD = 10112
RPT = N_PAD // NS
EPW = E // NW
K = 128
NBUF = 2
IB = 16
ITERS = IB * (-(-E // (NW * K * IB)))
E_PAD = NW * K * ITERS

_mesh = plsc.VectorSubcoreMesh(core_axis_name="c", subcore_axis_name="s")


@functools.partial(
    pl.kernel,
    out_type=jax.ShapeDtypeStruct((NW, N_PAD), jnp.float32),
    mesh=_mesh,
    scratch_types=[
        pltpu.VMEM((EPW,), jnp.int32),
        pltpu.VMEM((N_PAD,), jnp.float32),
    ],
    compiler_params=pltpu.CompilerParams(needs_layout_passes=False),
)
def _deg_kernel(dst_hbm, out_hbm, dst_v, hist_v):
    c = lax.axis_index("c")
    s = lax.axis_index("s")
    wid = s * NC + c
    pltpu.sync_copy(dst_hbm.at[pl.ds(wid * EPW, EPW)], dst_v)

    zeros = jnp.zeros((L,), jnp.float32)

    def zbody(i, carry):
        hist_v[pl.ds(i * L, L)] = zeros
        return carry

    lax.fori_loop(0, N_PAD // L, zbody, 0)

    ones = jnp.ones((L,), jnp.float32)

    def body(i, carry):
        idx = dst_v[pl.ds(i * L, L)]
        plsc.addupdate_scatter(hist_v, [idx], ones)
        return carry

    lax.fori_loop(0, EPW // L, body, 0)
    pltpu.sync_copy(hist_v, out_hbm.at[wid])


@functools.partial(
    pl.kernel,
    out_type=jax.ShapeDtypeStruct((NC, N_PAD, H), jnp.float32),
    mesh=_mesh,
    scratch_types=[
        pltpu.VMEM_SHARED((N_PAD, H), jnp.float32),
        pltpu.VMEM((ITERS, K), jnp.int32),
        pltpu.VMEM((2, IB, K), jnp.int32),
        pltpu.VMEM((NBUF, K, H), jnp.float32),
        pltpu.SemaphoreType.DMA,
        pltpu.SemaphoreType.DMA,
        pltpu.SemaphoreType.DMA,
        pltpu.SemaphoreType.DMA,
    ],
    compiler_params=pltpu.CompilerParams(needs_layout_passes=False,
                                         use_tc_tiling_on_sc=False),
)
def _agg_kernel(src_hbm, dst_hbm, y_hbm, zrows_hbm, out_hbm,
                acc_sh, src_v, dst_v, rows_v, gsem0, gsem1, dsem0, dsem1):
    c = lax.axis_index("c")
    s = lax.axis_index("s")
    wid = s * NC + c
    gsems = [gsem0, gsem1]
    dsems = [dsem0, dsem1]
    GI = IB // NBUF
    NBLK = ITERS // IB

    pltpu.sync_copy(zrows_hbm.at[pl.ds(s * RPT, RPT)],
                    acc_sh.at[pl.ds(s * RPT, RPT)])
    pltpu.sync_copy(src_hbm.at[wid], src_v)
    for blk in range(min(2, NBLK)):
        pltpu.async_copy(dst_hbm.at[wid, pl.ds(blk * IB, IB)],
                         dst_v.at[blk], dsems[blk])
    plsc.subcore_barrier()

    for b in range(NBUF):
        pltpu.async_copy(y_hbm.at[src_v.at[b]], rows_v.at[b], gsems[b])

    for blk in range(NBLK):
        buf = blk % 2
        pltpu.make_async_copy(dst_hbm.at[wid, pl.ds(blk * IB, IB)],
                              dst_v.at[buf], dsems[buf]).wait()

        def inner(g, carry2, blk=blk, buf=buf):
            for b in range(NBUF):
                j = blk * IB + g * NBUF + b
                pltpu.make_async_copy(y_hbm.at[src_v.at[j]],
                                      rows_v.at[b], gsems[b]).wait()
                pltpu.sync_copy(rows_v.at[b],
                                acc_sh.at[dst_v.at[buf].at[g * NBUF + b]],
                                add=True)

                @pl.when(j + NBUF < ITERS)
                def _():
                    pltpu.async_copy(y_hbm.at[src_v.at[j + NBUF]],
                                     rows_v.at[b], gsems[b])
            return carry2

        lax.fori_loop(0, GI, inner, 0)
        if blk + 2 < NBLK:
            pltpu.async_copy(dst_hbm.at[wid, pl.ds((blk + 2) * IB, IB)],
                             dst_v.at[buf], dsems[buf])
    plsc.subcore_barrier()

    pltpu.sync_copy(acc_sh.at[pl.ds(s * RPT, RPT)],
                    out_hbm.at[c, pl.ds(s * RPT, RPT)])


def _prescale_body(x_ref, w_ref, parts_ref, y_ref, dis_ref):
    deg = jnp.sum(parts_ref[...], axis=0)
    dis = jnp.where(deg > 0, lax.rsqrt(deg), 0.0)
    dis_col = dis.reshape(N_PAD, 1)
    dis_ref[...] = dis_col
    xw = jnp.dot(x_ref[...], w_ref[...], preferred_element_type=jnp.float32)
    y_ref[...] = dis_col * xw


def _nan_to_num(h):
    h = jnp.where(jnp.isnan(h), 0.0, h)
    h = jnp.where(h == jnp.inf, 100.0, h)
    h = jnp.where(h == -jnp.inf, -100.0, h)
    return h


def _mid_body(p_ref, dis_ref, x_ref, b1_ref, g1_ref, be1_ref, w2_ref,
              h1_ref, y2_ref):
    conv = dis_ref[...] * (p_ref[0] + p_ref[1]) + b1_ref[...]
    bnscale = 1.0 / jnp.sqrt(1.0 + EPS)
    h = conv * bnscale * g1_ref[...] + be1_ref[...]
    h = jax.nn.relu(h) + x_ref[...]
    h = _nan_to_num(h)
    h1_ref[...] = h
    y2_ref[...] = dis_ref[...] * jnp.dot(h, w2_ref[...],
                                         preferred_element_type=jnp.float32)


def _head_body(q_ref, dis_ref, h1_ref, b2_ref, g2_ref, be2_ref,
               wc1_ref, bc1_ref, wc2_ref, bc2_ref, logits_ref, probs_ref):
    conv = dis_ref[...] * (q_ref[0] + q_ref[1]) + b2_ref[...]
    bnscale = 1.0 / jnp.sqrt(1.0 + EPS)
    h = conv * bnscale * g2_ref[...] + be2_ref[...]
    h = h + h1_ref[...]
    h = _nan_to_num(h)
    hid = jax.nn.relu(jnp.dot(h, wc1_ref[...],
                              preferred_element_type=jnp.float32) + bc1_ref[...])
    logits = jnp.dot(hid, wc2_ref[...],
                     preferred_element_type=jnp.float32) + bc2_ref[...]
    logits = logits[:N]
    logits_ref[...] = logits
    m = jnp.max(logits, axis=1, keepdims=True)
    e = jnp.exp(logits - m)
    probs_ref[...] = e / jnp.sum(e, axis=1, keepdims=True)


def kernel(x, edge_indices, W1, b1, g1, be1, W2, b2, g2, be2, Wc1, bc1, Wc2, bc2):
    edge_index = edge_indices[0]
    src = edge_index[0]
    dst = edge_index[1]

    x_pad = jnp.pad(x, ((0, N_PAD - N), (0, 0)))
    pad_e = E_PAD - E
    pad_idx = (N + jnp.arange(pad_e, dtype=jnp.int32) % (N_PAD - N)).astype(jnp.int32)
    srcp = jnp.concatenate([src, pad_idx])
    dstp = jnp.concatenate([dst, pad_idx])
    srcp = srcp.reshape(NW, ITERS, K)
    dstp = dstp.reshape(NW, ITERS, K)
    zrows = jnp.zeros((N_PAD, H), jnp.float32)

    parts = _deg_kernel(dst)

    y1, dis_col = pl.pallas_call(
        _prescale_body,
        out_shape=(jax.ShapeDtypeStruct((N_PAD, H), jnp.float32),
                   jax.ShapeDtypeStruct((N_PAD, 1), jnp.float32)),
    )(x_pad, W1, parts)

    p = _agg_kernel(srcp, dstp, y1, zrows)

    h1, y2 = pl.pallas_call(
        _mid_body,
        out_shape=(jax.ShapeDtypeStruct((N_PAD, H), jnp.float32),
                   jax.ShapeDtypeStruct((N_PAD, H), jnp.float32)),
    )(p, dis_col, x_pad, b1, g1, be1, W2)

    q = _agg_kernel(srcp, dstp, y2, zrows)

    logits, probs = pl.pallas_call(
        _head_body,
        out_shape=(jax.ShapeDtypeStruct((N, 2), jnp.float32),
                   jax.ShapeDtypeStruct((N, 2), jnp.float32)),
    )(q, dis_col, h1, b2, g2, be2, Wc1, bc1, Wc2, bc2)

    return logits, probs


# speedup vs baseline: 1.1682x; 1.0832x over previous
import functools

import jax
import jax.numpy as jnp
from jax import lax
from jax.experimental import pallas as pl
from jax.experimental.pallas import tpu as pltpu
from jax.experimental.pallas import tpu_sc as plsc

N = 10000
E = 320000
D = 128
H = 128
EPS = 1e-5

NC = 2
NS = 16
NW = NC * NS
L = 16

N_PAD = 10112
RPT = N_PAD // NS
EPW = E // NW
K = 128
NBUF = 2
IB = 16
EB = EPW // K
TAIL = EPW - EB * K
DBLK = IB * K
NBLK = -(-EPW // DBLK)

_mesh = plsc.VectorSubcoreMesh(core_axis_name="c", subcore_axis_name="s")


@functools.partial(
    pl.kernel,
    out_type=jax.ShapeDtypeStruct((NW, N_PAD), jnp.float32),
    mesh=_mesh,
    scratch_types=[
        pltpu.VMEM((EPW,), jnp.int32),
        pltpu.VMEM((N_PAD,), jnp.float32),
    ],
    compiler_params=pltpu.CompilerParams(needs_layout_passes=False),
)
def _deg_kernel(edges_hbm, out_hbm, dst_v, hist_v):
    c = lax.axis_index("c")
    s = lax.axis_index("s")
    wid = s * NC + c
    pltpu.sync_copy(edges_hbm.at[pl.ds(E + wid * EPW, EPW)], dst_v)

    zeros = jnp.zeros((L,), jnp.float32)

    def zbody(i, carry):
        hist_v[pl.ds(i * L, L)] = zeros
        return carry

    lax.fori_loop(0, N_PAD // L, zbody, 0)

    ones = jnp.ones((L,), jnp.float32)

    def body(i, carry):
        idx = dst_v[pl.ds(i * L, L)]
        plsc.addupdate_scatter(hist_v, [idx], ones)
        return carry

    lax.fori_loop(0, EPW // L, body, 0)
    pltpu.sync_copy(hist_v, out_hbm.at[wid])


@functools.partial(
    pl.kernel,
    out_type=jax.ShapeDtypeStruct((NC, N_PAD, H), jnp.float32),
    mesh=_mesh,
    scratch_types=[
        pltpu.VMEM_SHARED((N_PAD, H), jnp.float32),
        pltpu.VMEM((EPW,), jnp.int32),
        pltpu.VMEM((2, DBLK), jnp.int32),
        pltpu.VMEM((NBUF, K, H), jnp.float32),
        pltpu.SemaphoreType.DMA,
        pltpu.SemaphoreType.DMA,
        pltpu.SemaphoreType.DMA,
        pltpu.SemaphoreType.DMA,
    ],
    compiler_params=pltpu.CompilerParams(needs_layout_passes=False,
                                         use_tc_tiling_on_sc=False),
)
def _agg_kernel(edges_hbm, y_hbm, zrows_hbm, out_hbm,
                acc_sh, src_v, dst_v, rows_v, gsem0, gsem1, dsem0, dsem1):
    c = lax.axis_index("c")
    s = lax.axis_index("s")
    wid = s * NC + c
    gsems = [gsem0, gsem1]
    dsems = [dsem0, dsem1]
    e0 = wid * EPW

    pltpu.sync_copy(zrows_hbm.at[pl.ds(s * RPT, RPT)],
                    acc_sh.at[pl.ds(s * RPT, RPT)])
    pltpu.sync_copy(edges_hbm.at[pl.ds(e0, EPW)], src_v)
    for blk in range(min(2, NBLK)):
        bsz = min(DBLK, EPW - blk * DBLK)
        pltpu.async_copy(edges_hbm.at[pl.ds(E + e0 + blk * DBLK, bsz)],
                         dst_v.at[blk, pl.ds(0, bsz)], dsems[blk])
    plsc.subcore_barrier()

    for b in range(NBUF):
        pltpu.async_copy(y_hbm.at[src_v.at[pl.ds(b * K, K)]],
                         rows_v.at[b], gsems[b])

    for blk in range(NBLK):
        buf = blk % 2
        bsz = min(DBLK, EPW - blk * DBLK)
        nb = min(IB, EB - blk * IB)
        pltpu.make_async_copy(edges_hbm.at[pl.ds(E + e0 + blk * DBLK, bsz)],
                              dst_v.at[buf, pl.ds(0, bsz)], dsems[buf]).wait()

        def inner(g, carry2, blk=blk, buf=buf):
            for b in range(NBUF):
                j = blk * IB + g * NBUF + b
                pltpu.make_async_copy(y_hbm.at[src_v.at[pl.ds(j * K, K)]],
                                      rows_v.at[b], gsems[b]).wait()
                lo = (g * NBUF + b) * K
                pltpu.sync_copy(rows_v.at[b],
                                acc_sh.at[dst_v.at[buf].at[pl.ds(lo, K)]],
                                add=True)

                @pl.when(j + NBUF < EB)
                def _():
                    jn = j + NBUF
                    pltpu.async_copy(y_hbm.at[src_v.at[pl.ds(jn * K, K)]],
                                     rows_v.at[b], gsems[b])
            return carry2

        lax.fori_loop(0, nb // NBUF, inner, 0)
        if blk + 2 < NBLK:
            nsz = min(DBLK, EPW - (blk + 2) * DBLK)
            pltpu.async_copy(
                edges_hbm.at[pl.ds(E + e0 + (blk + 2) * DBLK, nsz)],
                dst_v.at[buf, pl.ds(0, nsz)], dsems[buf])

    if TAIL:
        pltpu.sync_copy(y_hbm.at[src_v.at[pl.ds(EB * K, TAIL)]],
                        rows_v.at[0, pl.ds(0, TAIL)])
        tlo = EB * K - (NBLK - 1) * DBLK
        pltpu.sync_copy(
            rows_v.at[0, pl.ds(0, TAIL)],
            acc_sh.at[dst_v.at[(NBLK - 1) % 2].at[pl.ds(tlo, TAIL)]],
            add=True)
    plsc.subcore_barrier()

    pltpu.sync_copy(acc_sh.at[pl.ds(s * RPT, RPT)],
                    out_hbm.at[c, pl.ds(s * RPT, RPT)])


def _prescale_body(x_ref, w_ref, parts_ref, y_ref, dis_ref):
    deg = jnp.sum(parts_ref[...], axis=0)
    dis = jnp.where(deg > 0, lax.rsqrt(deg), 0.0)
    dis_col = dis[:N].reshape(N, 1)
    dis_ref[...] = dis_col
    xw = jnp.dot(x_ref[...], w_ref[...], preferred_element_type=jnp.float32)
    y_ref[...] = dis_col * xw


def _nan_to_num(h):
    h = jnp.where(jnp.isnan(h), 0.0, h)
    h = jnp.where(h == jnp.inf, 100.0, h)
    h = jnp.where(h == -jnp.inf, -100.0, h)
    return h


def _mid_body(p_ref, dis_ref, x_ref, b1_ref, g1_ref, be1_ref, w2_ref,
              h1_ref, y2_ref):
    agg = p_ref[0, :N] + p_ref[1, :N]
    conv = dis_ref[...] * agg + b1_ref[...]
    bnscale = 1.0 / jnp.sqrt(1.0 + EPS)
    h = conv * bnscale * g1_ref[...] + be1_ref[...]
    h = jax.nn.relu(h) + x_ref[...]
    h = _nan_to_num(h)
    h1_ref[...] = h
    y2_ref[...] = dis_ref[...] * jnp.dot(h, w2_ref[...],
                                         preferred_element_type=jnp.float32)


def _head_body(q_ref, dis_ref, h1_ref, b2_ref, g2_ref, be2_ref,
               wc1_ref, bc1_ref, wc2_ref, bc2_ref, logits_ref, probs_ref):
    agg = q_ref[0, :N] + q_ref[1, :N]
    conv = dis_ref[...] * agg + b2_ref[...]
    bnscale = 1.0 / jnp.sqrt(1.0 + EPS)
    h = conv * bnscale * g2_ref[...] + be2_ref[...]
    h = h + h1_ref[...]
    h = _nan_to_num(h)
    hid = jax.nn.relu(jnp.dot(h, wc1_ref[...],
                              preferred_element_type=jnp.float32) + bc1_ref[...])
    logits = jnp.dot(hid, wc2_ref[...],
                     preferred_element_type=jnp.float32) + bc2_ref[...]
    logits_ref[...] = logits
    m = jnp.max(logits, axis=1, keepdims=True)
    e = jnp.exp(logits - m)
    probs_ref[...] = e / jnp.sum(e, axis=1, keepdims=True)


def kernel(x, edge_indices, W1, b1, g1, be1, W2, b2, g2, be2, Wc1, bc1, Wc2, bc2):
    zrows = jnp.zeros((N_PAD, H), jnp.float32)
    eflat = edge_indices.reshape(2 * E)

    parts = _deg_kernel(eflat)

    y1, dis_col = pl.pallas_call(
        _prescale_body,
        out_shape=(jax.ShapeDtypeStruct((N, H), jnp.float32),
                   jax.ShapeDtypeStruct((N, 1), jnp.float32)),
    )(x, W1, parts)

    p = _agg_kernel(eflat, y1, zrows)

    h1, y2 = pl.pallas_call(
        _mid_body,
        out_shape=(jax.ShapeDtypeStruct((N, H), jnp.float32),
                   jax.ShapeDtypeStruct((N, H), jnp.float32)),
    )(p, dis_col, x, b1, g1, be1, W2)

    q = _agg_kernel(eflat, y2, zrows)

    logits, probs = pl.pallas_call(
        _head_body,
        out_shape=(jax.ShapeDtypeStruct((N, 2), jnp.float32),
                   jax.ShapeDtypeStruct((N, 2), jnp.float32)),
    )(q, dis_col, h1, b2, g2, be2, Wc1, bc1, Wc2, bc2)

    return logits, probs
